# baseline (speedup 1.0000x reference)
import functools

import jax
import jax.numpy as jnp
from jax import lax
from jax.experimental import pallas as pl
from jax.experimental.pallas import tpu as pltpu
from jax.experimental.pallas import tpu_sc as plsc

N = 10000
E = 320000
D = 128
VOCAB = 500
VPAD = 512
B = 64
D_OUT = 64

NC = 2
NS = 16
NW = NC * NS

NPAD = 10240
K = 64
NBUF = 4
CPT0 = 192
CPT1 = 128
EPHASE = 64
EPAD = NS * (CPT0 + CPT1) * K
NCHUNKS = EPAD // K
EDGES_PER_TILE_DEG = E // NW
ROWS_PER_TILE = NPAD // NS

BN = 1024
NB = NPAD // BN

F32 = jnp.float32


def _mesh():
    return plsc.VectorSubcoreMesh(core_axis_name="c", subcore_axis_name="s",
                                  num_cores=NC, num_subcores=NS)


@functools.cache
def _make_sc_deg():
    return pl.kernel(
        _sc_deg_body,
        out_type=jax.ShapeDtypeStruct((NW * NPAD,), F32),
        mesh=_mesh(),
        scratch_types=[
            pltpu.VMEM((EDGES_PER_TILE_DEG,), jnp.int32),
            pltpu.VMEM((NPAD,), F32),
        ],
        compiler_params=pltpu.CompilerParams(needs_layout_passes=False),
    )


def _sc_deg_body(dst_hbm, out_hbm, dstv, degloc):
    c = lax.axis_index("c")
    s = lax.axis_index("s")
    wid = c * NS + s
    zv = jnp.zeros((16,), F32)

    def zbody(i, carry):
        degloc[pl.ds(i * 16, 16)] = zv
        return carry

    lax.fori_loop(0, NPAD // 16, zbody, 0)

    pltpu.sync_copy(dst_hbm.at[pl.ds(wid * EDGES_PER_TILE_DEG,
                                     EDGES_PER_TILE_DEG)], dstv)
    ones = jnp.full((16,), 1.0, F32)

    def body(i, carry):
        idx = dstv[pl.ds(i * 16, 16)]
        plsc.addupdate_scatter(degloc, [idx], ones)
        return carry

    lax.fori_loop(0, EDGES_PER_TILE_DEG // 16, body, 0)
    pltpu.sync_copy(degloc, out_hbm.at[pl.ds(wid * NPAD, NPAD)])


@functools.cache
def _make_sc_edge():
    return pl.kernel(
        _sc_edge_body,
        out_type=jax.ShapeDtypeStruct((NC, NPAD, D), F32),
        mesh=_mesh(),
        scratch_types=[
            pltpu.VMEM((EPHASE, K), jnp.int32),
            pltpu.VMEM((EPHASE, K), jnp.int32),
            [pltpu.VMEM((K, D), F32) for _ in range(NBUF)],
            [pltpu.SemaphoreType.DMA for _ in range(NBUF)],
            pltpu.VMEM_SHARED((NPAD, D), F32),
        ],
        compiler_params=pltpu.CompilerParams(needs_layout_passes=False),
    )


def _sc_edge_body(g_hbm, src_hbm, dst_hbm, out_hbm, srcv, dstv, bufs,
                  gsems, acc):
    c = lax.axis_index("c")
    s = lax.axis_index("s")
    zv = jnp.zeros((16,), F32)
    buf0 = bufs[0]

    def zbody(i, carry):
        for j in range(D // 16):
            buf0[i, pl.ds(j * 16, 16)] = zv
        return carry

    lax.fori_loop(0, K, zbody, 0)
    row0 = s * ROWS_PER_TILE
    for k in range(ROWS_PER_TILE // K):
        pltpu.sync_copy(buf0, acc.at[pl.ds(row0 + k * K, K)])
    plsc.subcore_barrier()

    def pipeline(chunk0, cpt):
        for p in range(cpt // EPHASE):
            base = chunk0 + p * EPHASE
            pltpu.sync_copy(src_hbm.at[pl.ds(base, EPHASE)], srcv)
            pltpu.sync_copy(dst_hbm.at[pl.ds(base, EPHASE)], dstv)
            for b in range(NBUF):
                pltpu.async_copy(g_hbm.at[srcv.at[b]], bufs[b], gsems[b])

            def ebody(t, carry):
                for b in range(NBUF):
                    j = NBUF * t + b
                    jn = lax.rem(j + NBUF, EPHASE)
                    pltpu.make_async_copy(g_hbm.at[srcv.at[j]], bufs[b],
                                          gsems[b]).wait()
                    pltpu.sync_copy(bufs[b], acc.at[dstv.at[j]], add=True)
                    pltpu.async_copy(g_hbm.at[srcv.at[jn]], bufs[b], gsems[b])
                return carry

            lax.fori_loop(0, EPHASE // NBUF, ebody, 0)
            for b in range(NBUF):
                pltpu.make_async_copy(g_hbm.at[srcv.at[b]], bufs[b],
                                      gsems[b]).wait()

    @pl.when(c == 0)
    def _():
        pipeline(s * CPT0, CPT0)

    @pl.when(c == 1)
    def _():
        pipeline(NS * CPT0 + s * CPT1, CPT1)

    plsc.subcore_barrier()

    pltpu.sync_copy(acc.at[pl.ds(row0, ROWS_PER_TILE)],
                    out_hbm.at[c, pl.ds(row0, ROWS_PER_TILE)])


def _tck1_body(deg_ref, x_ref, emb_ref, w1_ref, g_ref, dinv_ref, embw_ref):
    i = pl.program_id(0)

    @pl.when(i == 0)
    def _():
        embw_ref[...] = jnp.dot(emb_ref[...], w1_ref[...],
                                preferred_element_type=F32)

    dn0 = (((0,), (0,)), ((), ()))
    deg = lax.dot_general(deg_ref[...], jnp.ones((NW, 1), F32), dn0,
                          preferred_element_type=F32) + 1.0
    dinv = lax.rsqrt(jnp.maximum(deg, 1.0))
    oh = (x_ref[...] == lax.broadcasted_iota(jnp.int32, (BN, VPAD), 1)
          ).astype(F32)
    hw = jnp.dot(oh, embw_ref[...], preferred_element_type=F32)
    rows = i * BN + lax.broadcasted_iota(jnp.int32, (BN, 1), 0)
    g_ref[...] = jnp.where(rows < N, dinv * hw, 0.0)
    dinv_ref[...] = dinv


_tck1 = pl.pallas_call(
    _tck1_body,
    grid=(NB,),
    in_specs=[
        pl.BlockSpec((NW, BN), lambda i: (0, i)),
        pl.BlockSpec((BN, 1), lambda i: (i, 0)),
        pl.BlockSpec((VPAD, D), lambda i: (0, 0)),
        pl.BlockSpec((D, D), lambda i: (0, 0)),
    ],
    out_specs=[
        pl.BlockSpec((BN, D), lambda i: (i, 0)),
        pl.BlockSpec((BN, 1), lambda i: (i, 0)),
    ],
    out_shape=[
        jax.ShapeDtypeStruct((NPAD, D), F32),
        jax.ShapeDtypeStruct((NPAD, 1), F32),
    ],
    scratch_shapes=[pltpu.VMEM((VPAD, D), F32)],
)


def _tck2_body(acc_ref, g1_ref, dinv_ref, b1_ref, w2_ref, g2_ref):
    i = pl.program_id(0)
    dinv = dinv_ref[...]
    a = acc_ref[0] + acc_ref[1] + g1_ref[...]
    h1 = jnp.maximum(dinv * a + b1_ref[...], 0.0)
    g2 = dinv * jnp.dot(h1, w2_ref[...], preferred_element_type=F32)
    rows = i * BN + lax.broadcasted_iota(jnp.int32, (BN, 1), 0)
    g2_ref[...] = jnp.where(rows < N, g2, 0.0)


_tck2 = pl.pallas_call(
    _tck2_body,
    grid=(NB,),
    in_specs=[
        pl.BlockSpec((NC, BN, D), lambda i: (0, i, 0)),
        pl.BlockSpec((BN, D), lambda i: (i, 0)),
        pl.BlockSpec((BN, 1), lambda i: (i, 0)),
        pl.BlockSpec((1, D), lambda i: (0, 0)),
        pl.BlockSpec((D, D), lambda i: (0, 0)),
    ],
    out_specs=pl.BlockSpec((BN, D), lambda i: (i, 0)),
    out_shape=jax.ShapeDtypeStruct((NPAD, D), F32),
)


def _tck3_body(acc_ref, g2_ref, dinv_ref, b2_ref, batch_ref, fcw_ref, fcb_ref,
               out_ref, sums_ref, cnts_ref):
    i = pl.program_id(0)

    @pl.when(i == 0)
    def _():
        sums_ref[...] = jnp.zeros_like(sums_ref)
        cnts_ref[...] = jnp.zeros_like(cnts_ref)

    dinv = dinv_ref[...]
    out2 = dinv * (acc_ref[0] + acc_ref[1] + g2_ref[...]) + b2_ref[...]
    oh = (batch_ref[...] == lax.broadcasted_iota(jnp.int32, (BN, B), 1)
          ).astype(F32)
    dn = (((0,), (0,)), ((), ()))
    sums_ref[...] += lax.dot_general(oh, out2, dn, preferred_element_type=F32)
    cnts_ref[...] += lax.dot_general(oh, jnp.ones((BN, D), F32), dn,
                                     preferred_element_type=F32)

    @pl.when(i == NB - 1)
    def _():
        pooled = sums_ref[...] / jnp.maximum(cnts_ref[...], 1.0)
        out_ref[...] = jnp.dot(pooled, fcw_ref[...],
                               preferred_element_type=F32) + fcb_ref[...]


_tck3 = pl.pallas_call(
    _tck3_body,
    grid=(NB,),
    in_specs=[
        pl.BlockSpec((NC, BN, D), lambda i: (0, i, 0)),
        pl.BlockSpec((BN, D), lambda i: (i, 0)),
        pl.BlockSpec((BN, 1), lambda i: (i, 0)),
        pl.BlockSpec((1, D), lambda i: (0, 0)),
        pl.BlockSpec((BN, 1), lambda i: (i, 0)),
        pl.BlockSpec((D, D_OUT), lambda i: (0, 0)),
        pl.BlockSpec((1, D_OUT), lambda i: (0, 0)),
    ],
    out_specs=pl.BlockSpec((B, D_OUT), lambda i: (0, 0)),
    out_shape=jax.ShapeDtypeStruct((B, D_OUT), F32),
    scratch_shapes=[pltpu.VMEM((B, D), F32), pltpu.VMEM((B, D), F32)],
)


def kernel(x, edge_index, batch, emb, W1, b1, W2, b2, fcW, fcb):
    x = x.astype(jnp.int32)
    edge_index = edge_index.astype(jnp.int32)
    batch = batch.astype(jnp.int32)

    src = edge_index[0]
    dst = edge_index[1]
    pad_src = jnp.full((EPAD - E,), N, jnp.int32)
    pad_dst = N + jnp.arange(EPAD - E, dtype=jnp.int32) % (NPAD - N)
    srcp = jnp.concatenate([src, pad_src]).reshape(NCHUNKS, K)
    dstp = jnp.concatenate([dst, pad_dst]).reshape(NCHUNKS, K)

    x3 = jnp.concatenate([x, jnp.zeros((NPAD - N,), jnp.int32)]
                         ).reshape(NPAD, 1)
    batch3 = jnp.concatenate([batch, jnp.full((NPAD - N,), B, jnp.int32)]
                             ).reshape(NPAD, 1)
    emb_pad = jnp.zeros((VPAD, D), F32).at[:VOCAB].set(emb)

    sc_deg = _make_sc_deg()
    sc_edge = _make_sc_edge()
    deg_parts = sc_deg(dst).reshape(NW, NPAD)
    g1, dinv3 = _tck1(deg_parts, x3, emb_pad, W1)
    acc1 = sc_edge(g1, srcp, dstp)
    g2 = _tck2(acc1, g1, dinv3, b1.reshape(1, D), W2)
    acc2 = sc_edge(g2, srcp, dstp)
    out = _tck3(acc2, g2, dinv3, b2.reshape(1, D), batch3, fcW,
                fcb.reshape(1, D_OUT))
    return out


# speedup vs baseline: 1.0187x; 1.0187x over previous
import functools

import jax
import jax.numpy as jnp
from jax import lax
from jax.experimental import pallas as pl
from jax.experimental.pallas import tpu as pltpu
from jax.experimental.pallas import tpu_sc as plsc

N = 10000
E = 320000
D = 128
VOCAB = 500
VPAD = 512
B = 64
D_OUT = 64

NC = 2
NS = 16
NW = NC * NS

NPAD = 10240
K = 64
NBUF = 4
CPT0 = 256
CPT1 = 64
EPHASE = 64
EPAD = NS * (CPT0 + CPT1) * K
NCHUNKS = EPAD // K
EDGES_PER_TILE_DEG = E // NW
ROWS_PER_TILE = NPAD // NS

BN = 1024
NB = NPAD // BN

F32 = jnp.float32


def _mesh():
    return plsc.VectorSubcoreMesh(core_axis_name="c", subcore_axis_name="s",
                                  num_cores=NC, num_subcores=NS)


@functools.cache
def _make_sc_deg():
    return pl.kernel(
        _sc_deg_body,
        out_type=jax.ShapeDtypeStruct((NW * NPAD,), F32),
        mesh=_mesh(),
        scratch_types=[
            pltpu.VMEM((EDGES_PER_TILE_DEG,), jnp.int32),
            pltpu.VMEM((NPAD,), F32),
        ],
        compiler_params=pltpu.CompilerParams(needs_layout_passes=False),
    )


def _sc_deg_body(dst_hbm, out_hbm, dstv, degloc):
    c = lax.axis_index("c")
    s = lax.axis_index("s")
    wid = c * NS + s
    zv = jnp.zeros((16,), F32)

    def zbody(i, carry):
        degloc[pl.ds(i * 16, 16)] = zv
        return carry

    lax.fori_loop(0, NPAD // 16, zbody, 0)

    pltpu.sync_copy(dst_hbm.at[pl.ds(wid * EDGES_PER_TILE_DEG,
                                     EDGES_PER_TILE_DEG)], dstv)
    ones = jnp.full((16,), 1.0, F32)

    def body(i, carry):
        idx = dstv[pl.ds(i * 16, 16)]
        plsc.addupdate_scatter(degloc, [idx], ones)
        return carry

    lax.fori_loop(0, EDGES_PER_TILE_DEG // 16, body, 0)
    pltpu.sync_copy(degloc, out_hbm.at[pl.ds(wid * NPAD, NPAD)])


@functools.cache
def _make_sc_edge():
    return pl.kernel(
        _sc_edge_body,
        out_type=jax.ShapeDtypeStruct((NC, NPAD, D), F32),
        mesh=_mesh(),
        scratch_types=[
            pltpu.VMEM((EPHASE, K), jnp.int32),
            pltpu.VMEM((EPHASE, K), jnp.int32),
            [pltpu.VMEM((K, D), F32) for _ in range(NBUF)],
            [pltpu.SemaphoreType.DMA for _ in range(NBUF)],
            pltpu.VMEM_SHARED((NPAD, D), F32),
        ],
        compiler_params=pltpu.CompilerParams(needs_layout_passes=False),
    )


def _sc_edge_body(g_hbm, src_hbm, dst_hbm, out_hbm, srcv, dstv, bufs,
                  gsems, acc):
    c = lax.axis_index("c")
    s = lax.axis_index("s")
    zv = jnp.zeros((16,), F32)
    buf0 = bufs[0]

    def zbody(i, carry):
        for j in range(D // 16):
            buf0[i, pl.ds(j * 16, 16)] = zv
        return carry

    lax.fori_loop(0, K, zbody, 0)
    row0 = s * ROWS_PER_TILE
    for k in range(ROWS_PER_TILE // K):
        pltpu.sync_copy(buf0, acc.at[pl.ds(row0 + k * K, K)])
    plsc.subcore_barrier()

    def pipeline(chunk0, cpt):
        for p in range(cpt // EPHASE):
            base = chunk0 + p * EPHASE
            pltpu.sync_copy(src_hbm.at[pl.ds(base, EPHASE)], srcv)
            pltpu.sync_copy(dst_hbm.at[pl.ds(base, EPHASE)], dstv)
            for b in range(NBUF):
                pltpu.async_copy(g_hbm.at[srcv.at[b]], bufs[b], gsems[b])

            def ebody(t, carry):
                for b in range(NBUF):
                    j = NBUF * t + b
                    jn = lax.rem(j + NBUF, EPHASE)
                    pltpu.make_async_copy(g_hbm.at[srcv.at[j]], bufs[b],
                                          gsems[b]).wait()
                    pltpu.sync_copy(bufs[b], acc.at[dstv.at[j]], add=True)
                    pltpu.async_copy(g_hbm.at[srcv.at[jn]], bufs[b], gsems[b])
                return carry

            lax.fori_loop(0, EPHASE // NBUF, ebody, 0)
            for b in range(NBUF):
                pltpu.make_async_copy(g_hbm.at[srcv.at[b]], bufs[b],
                                      gsems[b]).wait()

    @pl.when(c == 0)
    def _():
        pipeline(s * CPT0, CPT0)

    @pl.when(c == 1)
    def _():
        pipeline(NS * CPT0 + s * CPT1, CPT1)

    plsc.subcore_barrier()

    pltpu.sync_copy(acc.at[pl.ds(row0, ROWS_PER_TILE)],
                    out_hbm.at[c, pl.ds(row0, ROWS_PER_TILE)])


def _tck1_body(deg_ref, x_ref, emb_ref, w1_ref, g_ref, dinv_ref, embw_ref):
    i = pl.program_id(0)

    @pl.when(i == 0)
    def _():
        embw_ref[...] = jnp.dot(emb_ref[...], w1_ref[...],
                                preferred_element_type=F32)

    dn0 = (((0,), (0,)), ((), ()))
    deg = lax.dot_general(deg_ref[...], jnp.ones((NW, 1), F32), dn0,
                          preferred_element_type=F32) + 1.0
    dinv = lax.rsqrt(jnp.maximum(deg, 1.0))
    oh = (x_ref[...] == lax.broadcasted_iota(jnp.int32, (BN, VPAD), 1)
          ).astype(F32)
    hw = jnp.dot(oh, embw_ref[...], preferred_element_type=F32)
    rows = i * BN + lax.broadcasted_iota(jnp.int32, (BN, 1), 0)
    g_ref[...] = jnp.where(rows < N, dinv * hw, 0.0)
    dinv_ref[...] = dinv


_tck1 = pl.pallas_call(
    _tck1_body,
    grid=(NB,),
    in_specs=[
        pl.BlockSpec((NW, BN), lambda i: (0, i)),
        pl.BlockSpec((BN, 1), lambda i: (i, 0)),
        pl.BlockSpec((VPAD, D), lambda i: (0, 0)),
        pl.BlockSpec((D, D), lambda i: (0, 0)),
    ],
    out_specs=[
        pl.BlockSpec((BN, D), lambda i: (i, 0)),
        pl.BlockSpec((BN, 1), lambda i: (i, 0)),
    ],
    out_shape=[
        jax.ShapeDtypeStruct((NPAD, D), F32),
        jax.ShapeDtypeStruct((NPAD, 1), F32),
    ],
    scratch_shapes=[pltpu.VMEM((VPAD, D), F32)],
)


def _tck2_body(acc_ref, g1_ref, dinv_ref, b1_ref, w2_ref, g2_ref):
    i = pl.program_id(0)
    dinv = dinv_ref[...]
    a = acc_ref[0] + acc_ref[1] + g1_ref[...]
    h1 = jnp.maximum(dinv * a + b1_ref[...], 0.0)
    g2 = dinv * jnp.dot(h1, w2_ref[...], preferred_element_type=F32)
    rows = i * BN + lax.broadcasted_iota(jnp.int32, (BN, 1), 0)
    g2_ref[...] = jnp.where(rows < N, g2, 0.0)


_tck2 = pl.pallas_call(
    _tck2_body,
    grid=(NB,),
    in_specs=[
        pl.BlockSpec((NC, BN, D), lambda i: (0, i, 0)),
        pl.BlockSpec((BN, D), lambda i: (i, 0)),
        pl.BlockSpec((BN, 1), lambda i: (i, 0)),
        pl.BlockSpec((1, D), lambda i: (0, 0)),
        pl.BlockSpec((D, D), lambda i: (0, 0)),
    ],
    out_specs=pl.BlockSpec((BN, D), lambda i: (i, 0)),
    out_shape=jax.ShapeDtypeStruct((NPAD, D), F32),
)


def _tck3_body(acc_ref, g2_ref, dinv_ref, b2_ref, batch_ref, fcw_ref, fcb_ref,
               out_ref, sums_ref, cnts_ref):
    i = pl.program_id(0)

    @pl.when(i == 0)
    def _():
        sums_ref[...] = jnp.zeros_like(sums_ref)
        cnts_ref[...] = jnp.zeros_like(cnts_ref)

    dinv = dinv_ref[...]
    out2 = dinv * (acc_ref[0] + acc_ref[1] + g2_ref[...]) + b2_ref[...]
    oh = (batch_ref[...] == lax.broadcasted_iota(jnp.int32, (BN, B), 1)
          ).astype(F32)
    dn = (((0,), (0,)), ((), ()))
    sums_ref[...] += lax.dot_general(oh, out2, dn, preferred_element_type=F32)
    cnts_ref[...] += lax.dot_general(oh, jnp.ones((BN, D), F32), dn,
                                     preferred_element_type=F32)

    @pl.when(i == NB - 1)
    def _():
        pooled = sums_ref[...] / jnp.maximum(cnts_ref[...], 1.0)
        out_ref[...] = jnp.dot(pooled, fcw_ref[...],
                               preferred_element_type=F32) + fcb_ref[...]


_tck3 = pl.pallas_call(
    _tck3_body,
    grid=(NB,),
    in_specs=[
        pl.BlockSpec((NC, BN, D), lambda i: (0, i, 0)),
        pl.BlockSpec((BN, D), lambda i: (i, 0)),
        pl.BlockSpec((BN, 1), lambda i: (i, 0)),
        pl.BlockSpec((1, D), lambda i: (0, 0)),
        pl.BlockSpec((BN, 1), lambda i: (i, 0)),
        pl.BlockSpec((D, D_OUT), lambda i: (0, 0)),
        pl.BlockSpec((1, D_OUT), lambda i: (0, 0)),
    ],
    out_specs=pl.BlockSpec((B, D_OUT), lambda i: (0, 0)),
    out_shape=jax.ShapeDtypeStruct((B, D_OUT), F32),
    scratch_shapes=[pltpu.VMEM((B, D), F32), pltpu.VMEM((B, D), F32)],
)


def kernel(x, edge_index, batch, emb, W1, b1, W2, b2, fcW, fcb):
    x = x.astype(jnp.int32)
    edge_index = edge_index.astype(jnp.int32)
    batch = batch.astype(jnp.int32)

    src = edge_index[0]
    dst = edge_index[1]
    pad_src = jnp.full((EPAD - E,), N, jnp.int32)
    pad_dst = N + jnp.arange(EPAD - E, dtype=jnp.int32) % (NPAD - N)
    srcp = jnp.concatenate([src, pad_src]).reshape(NCHUNKS, K)
    dstp = jnp.concatenate([dst, pad_dst]).reshape(NCHUNKS, K)

    x3 = jnp.concatenate([x, jnp.zeros((NPAD - N,), jnp.int32)]
                         ).reshape(NPAD, 1)
    batch3 = jnp.concatenate([batch, jnp.full((NPAD - N,), B, jnp.int32)]
                             ).reshape(NPAD, 1)
    emb_pad = jnp.zeros((VPAD, D), F32).at[:VOCAB].set(emb)

    sc_deg = _make_sc_deg()
    sc_edge = _make_sc_edge()
    deg_parts = sc_deg(dst).reshape(NW, NPAD)
    g1, dinv3 = _tck1(deg_parts, x3, emb_pad, W1)
    acc1 = sc_edge(g1, srcp, dstp)
    g2 = _tck2(acc1, g1, dinv3, b1.reshape(1, D), W2)
    acc2 = sc_edge(g2, srcp, dstp)
    out = _tck3(acc2, g2, dinv3, b2.reshape(1, D), batch3, fcW,
                fcb.reshape(1, D_OUT))
    return out


# speedup vs baseline: 1.0253x; 1.0065x over previous
import functools

import jax
import jax.numpy as jnp
from jax import lax
from jax.experimental import pallas as pl
from jax.experimental.pallas import tpu as pltpu
from jax.experimental.pallas import tpu_sc as plsc

N = 10000
E = 320000
D = 128
VOCAB = 500
VPAD = 512
B = 64
D_OUT = 64

NC = 2
NS = 16
NW = NC * NS

NPAD = 10240
K = 64
NBUF = 4
CPT0 = 288
CPT1 = 32
EPHASE = 32
EPAD = NS * (CPT0 + CPT1) * K
NCHUNKS = EPAD // K
EDGES_PER_TILE_DEG = E // NW
ROWS_PER_TILE = NPAD // NS

BN = 1024
NB = NPAD // BN

F32 = jnp.float32


def _mesh():
    return plsc.VectorSubcoreMesh(core_axis_name="c", subcore_axis_name="s",
                                  num_cores=NC, num_subcores=NS)


@functools.cache
def _make_sc_deg():
    return pl.kernel(
        _sc_deg_body,
        out_type=jax.ShapeDtypeStruct((NW * NPAD,), F32),
        mesh=_mesh(),
        scratch_types=[
            pltpu.VMEM((EDGES_PER_TILE_DEG,), jnp.int32),
            pltpu.VMEM((NPAD,), F32),
        ],
        compiler_params=pltpu.CompilerParams(needs_layout_passes=False),
    )


def _sc_deg_body(dst_hbm, out_hbm, dstv, degloc):
    c = lax.axis_index("c")
    s = lax.axis_index("s")
    wid = c * NS + s
    zv = jnp.zeros((16,), F32)

    def zbody(i, carry):
        degloc[pl.ds(i * 16, 16)] = zv
        return carry

    lax.fori_loop(0, NPAD // 16, zbody, 0)

    pltpu.sync_copy(dst_hbm.at[pl.ds(wid * EDGES_PER_TILE_DEG,
                                     EDGES_PER_TILE_DEG)], dstv)
    ones = jnp.full((16,), 1.0, F32)

    def body(i, carry):
        idx = dstv[pl.ds(i * 16, 16)]
        plsc.addupdate_scatter(degloc, [idx], ones)
        return carry

    lax.fori_loop(0, EDGES_PER_TILE_DEG // 16, body, 0)
    pltpu.sync_copy(degloc, out_hbm.at[pl.ds(wid * NPAD, NPAD)])


@functools.cache
def _make_sc_edge():
    return pl.kernel(
        _sc_edge_body,
        out_type=jax.ShapeDtypeStruct((NC, NPAD, D), F32),
        mesh=_mesh(),
        scratch_types=[
            pltpu.VMEM((EPHASE, K), jnp.int32),
            pltpu.VMEM((EPHASE, K), jnp.int32),
            [pltpu.VMEM((K, D), F32) for _ in range(NBUF)],
            [pltpu.SemaphoreType.DMA for _ in range(NBUF)],
            pltpu.VMEM_SHARED((NPAD, D), F32),
        ],
        compiler_params=pltpu.CompilerParams(needs_layout_passes=False),
    )


def _sc_edge_body(g_hbm, src_hbm, dst_hbm, out_hbm, srcv, dstv, bufs,
                  gsems, acc):
    c = lax.axis_index("c")
    s = lax.axis_index("s")
    zv = jnp.zeros((16,), F32)
    buf0 = bufs[0]

    def zbody(i, carry):
        for j in range(D // 16):
            buf0[i, pl.ds(j * 16, 16)] = zv
        return carry

    lax.fori_loop(0, K, zbody, 0)
    row0 = s * ROWS_PER_TILE
    for k in range(ROWS_PER_TILE // K):
        pltpu.sync_copy(buf0, acc.at[pl.ds(row0 + k * K, K)])
    plsc.subcore_barrier()

    def pipeline(chunk0, cpt):
        for p in range(cpt // EPHASE):
            base = chunk0 + p * EPHASE
            pltpu.sync_copy(src_hbm.at[pl.ds(base, EPHASE)], srcv)
            pltpu.sync_copy(dst_hbm.at[pl.ds(base, EPHASE)], dstv)
            for b in range(NBUF):
                pltpu.async_copy(g_hbm.at[srcv.at[b]], bufs[b], gsems[b])

            def ebody(t, carry):
                for b in range(NBUF):
                    j = NBUF * t + b
                    jn = lax.rem(j + NBUF, EPHASE)
                    pltpu.make_async_copy(g_hbm.at[srcv.at[j]], bufs[b],
                                          gsems[b]).wait()
                    pltpu.sync_copy(bufs[b], acc.at[dstv.at[j]], add=True)
                    pltpu.async_copy(g_hbm.at[srcv.at[jn]], bufs[b], gsems[b])
                return carry

            lax.fori_loop(0, EPHASE // NBUF, ebody, 0)
            for b in range(NBUF):
                pltpu.make_async_copy(g_hbm.at[srcv.at[b]], bufs[b],
                                      gsems[b]).wait()

    @pl.when(c == 0)
    def _():
        pipeline(s * CPT0, CPT0)

    @pl.when(c == 1)
    def _():
        pipeline(NS * CPT0 + s * CPT1, CPT1)

    plsc.subcore_barrier()

    pltpu.sync_copy(acc.at[pl.ds(row0, ROWS_PER_TILE)],
                    out_hbm.at[c, pl.ds(row0, ROWS_PER_TILE)])


def _tck1_body(deg_ref, x_ref, emb_ref, w1_ref, g_ref, dinv_ref, embw_ref):
    i = pl.program_id(0)

    @pl.when(i == 0)
    def _():
        embw_ref[...] = jnp.dot(emb_ref[...], w1_ref[...],
                                preferred_element_type=F32)

    dn0 = (((0,), (0,)), ((), ()))
    deg = lax.dot_general(deg_ref[...], jnp.ones((NW, 1), F32), dn0,
                          preferred_element_type=F32) + 1.0
    dinv = lax.rsqrt(jnp.maximum(deg, 1.0))
    oh = (x_ref[...] == lax.broadcasted_iota(jnp.int32, (BN, VPAD), 1)
          ).astype(F32)
    hw = jnp.dot(oh, embw_ref[...], preferred_element_type=F32)
    rows = i * BN + lax.broadcasted_iota(jnp.int32, (BN, 1), 0)
    g_ref[...] = jnp.where(rows < N, dinv * hw, 0.0)
    dinv_ref[...] = dinv


_tck1 = pl.pallas_call(
    _tck1_body,
    grid=(NB,),
    in_specs=[
        pl.BlockSpec((NW, BN), lambda i: (0, i)),
        pl.BlockSpec((BN, 1), lambda i: (i, 0)),
        pl.BlockSpec((VPAD, D), lambda i: (0, 0)),
        pl.BlockSpec((D, D), lambda i: (0, 0)),
    ],
    out_specs=[
        pl.BlockSpec((BN, D), lambda i: (i, 0)),
        pl.BlockSpec((BN, 1), lambda i: (i, 0)),
    ],
    out_shape=[
        jax.ShapeDtypeStruct((NPAD, D), F32),
        jax.ShapeDtypeStruct((NPAD, 1), F32),
    ],
    scratch_shapes=[pltpu.VMEM((VPAD, D), F32)],
)


def _tck2_body(acc_ref, g1_ref, dinv_ref, b1_ref, w2_ref, g2_ref):
    i = pl.program_id(0)
    dinv = dinv_ref[...]
    a = acc_ref[0] + acc_ref[1] + g1_ref[...]
    h1 = jnp.maximum(dinv * a + b1_ref[...], 0.0)
    g2 = dinv * jnp.dot(h1, w2_ref[...], preferred_element_type=F32)
    rows = i * BN + lax.broadcasted_iota(jnp.int32, (BN, 1), 0)
    g2_ref[...] = jnp.where(rows < N, g2, 0.0)


_tck2 = pl.pallas_call(
    _tck2_body,
    grid=(NB,),
    in_specs=[
        pl.BlockSpec((NC, BN, D), lambda i: (0, i, 0)),
        pl.BlockSpec((BN, D), lambda i: (i, 0)),
        pl.BlockSpec((BN, 1), lambda i: (i, 0)),
        pl.BlockSpec((1, D), lambda i: (0, 0)),
        pl.BlockSpec((D, D), lambda i: (0, 0)),
    ],
    out_specs=pl.BlockSpec((BN, D), lambda i: (i, 0)),
    out_shape=jax.ShapeDtypeStruct((NPAD, D), F32),
)


def _tck3_body(acc_ref, g2_ref, dinv_ref, b2_ref, batch_ref, fcw_ref, fcb_ref,
               out_ref, sums_ref, cnts_ref):
    i = pl.program_id(0)

    @pl.when(i == 0)
    def _():
        sums_ref[...] = jnp.zeros_like(sums_ref)
        cnts_ref[...] = jnp.zeros_like(cnts_ref)

    dinv = dinv_ref[...]
    out2 = dinv * (acc_ref[0] + acc_ref[1] + g2_ref[...]) + b2_ref[...]
    oh = (batch_ref[...] == lax.broadcasted_iota(jnp.int32, (BN, B), 1)
          ).astype(F32)
    dn = (((0,), (0,)), ((), ()))
    sums_ref[...] += lax.dot_general(oh, out2, dn, preferred_element_type=F32)
    cnts_ref[...] += lax.dot_general(oh, jnp.ones((BN, D), F32), dn,
                                     preferred_element_type=F32)

    @pl.when(i == NB - 1)
    def _():
        pooled = sums_ref[...] / jnp.maximum(cnts_ref[...], 1.0)
        out_ref[...] = jnp.dot(pooled, fcw_ref[...],
                               preferred_element_type=F32) + fcb_ref[...]


_tck3 = pl.pallas_call(
    _tck3_body,
    grid=(NB,),
    in_specs=[
        pl.BlockSpec((NC, BN, D), lambda i: (0, i, 0)),
        pl.BlockSpec((BN, D), lambda i: (i, 0)),
        pl.BlockSpec((BN, 1), lambda i: (i, 0)),
        pl.BlockSpec((1, D), lambda i: (0, 0)),
        pl.BlockSpec((BN, 1), lambda i: (i, 0)),
        pl.BlockSpec((D, D_OUT), lambda i: (0, 0)),
        pl.BlockSpec((1, D_OUT), lambda i: (0, 0)),
    ],
    out_specs=pl.BlockSpec((B, D_OUT), lambda i: (0, 0)),
    out_shape=jax.ShapeDtypeStruct((B, D_OUT), F32),
    scratch_shapes=[pltpu.VMEM((B, D), F32), pltpu.VMEM((B, D), F32)],
)


def kernel(x, edge_index, batch, emb, W1, b1, W2, b2, fcW, fcb):
    x = x.astype(jnp.int32)
    edge_index = edge_index.astype(jnp.int32)
    batch = batch.astype(jnp.int32)

    src = edge_index[0]
    dst = edge_index[1]
    pad_src = jnp.full((EPAD - E,), N, jnp.int32)
    pad_dst = N + jnp.arange(EPAD - E, dtype=jnp.int32) % (NPAD - N)
    srcp = jnp.concatenate([src, pad_src]).reshape(NCHUNKS, K)
    dstp = jnp.concatenate([dst, pad_dst]).reshape(NCHUNKS, K)

    x3 = jnp.concatenate([x, jnp.zeros((NPAD - N,), jnp.int32)]
                         ).reshape(NPAD, 1)
    batch3 = jnp.concatenate([batch, jnp.full((NPAD - N,), B, jnp.int32)]
                             ).reshape(NPAD, 1)
    emb_pad = jnp.zeros((VPAD, D), F32).at[:VOCAB].set(emb)

    sc_deg = _make_sc_deg()
    sc_edge = _make_sc_edge()
    deg_parts = sc_deg(dst).reshape(NW, NPAD)
    g1, dinv3 = _tck1(deg_parts, x3, emb_pad, W1)
    acc1 = sc_edge(g1, srcp, dstp)
    g2 = _tck2(acc1, g1, dinv3, b1.reshape(1, D), W2)
    acc2 = sc_edge(g2, srcp, dstp)
    out = _tck3(acc2, g2, dinv3, b2.reshape(1, D), batch3, fcW,
                fcb.reshape(1, D_OUT))
    return out
